# Initial kernel scaffold; baseline (speedup 1.0000x reference)
import jax
import jax.numpy as jnp
from jax.experimental import pallas as pl


def kernel(x, edge_index, W1, b1, W2, b2):
    raise NotImplementedError("write your pallas kernel here")


# speedup vs baseline: 50.1151x; 50.1151x over previous
import functools

import jax
import jax.numpy as jnp
from jax import lax
from jax.experimental import pallas as pl
from jax.experimental.pallas import tpu as pltpu
from jax.experimental.pallas import tpu_sc as plsc

N = 10000
E = 320000
D = 128
H = 16
C = 2

NC = 2
NS = 16
NW = NC * NS
EPT = E // NW
CH = 2000
NCHUNK = EPT // CH
NPAD = 10240
ZCH = NPAD // NS
RPT = NPAD // NS

_MESH = plsc.VectorSubcoreMesh(core_axis_name="c", subcore_axis_name="s")


def _deg_body(dst_hbm, out_hbm, idx_v, ones_v, zer_v, acc_sh, sem):
  del sem
  cid = lax.axis_index("c")
  sid = lax.axis_index("s")
  wid = cid * NS + sid

  def fill_ones(i, _):
    ones_v[pl.ds(i * 16, 16)] = jnp.ones((16,), jnp.float32)
    return 0

  def fill_zeros(i, _):
    zer_v[pl.ds(i * 16, 16)] = jnp.zeros((16,), jnp.float32)
    return 0

  lax.fori_loop(0, CH // 16, fill_ones, 0)
  lax.fori_loop(0, ZCH // 16, fill_zeros, 0)

  pltpu.sync_copy(zer_v, acc_sh.at[pl.ds(sid * ZCH, ZCH)])
  plsc.subcore_barrier()

  base = wid * EPT
  for k in range(NCHUNK):
    pltpu.sync_copy(dst_hbm.at[pl.ds(base + k * CH, CH)], idx_v)
    pltpu.sync_copy(ones_v, acc_sh.at[idx_v], add=True)

  plsc.subcore_barrier()
  pltpu.sync_copy(acc_sh.at[pl.ds(sid * ZCH, ZCH)],
                  out_hbm.at[pl.ds(cid * NPAD + sid * ZCH, ZCH)])


_deg_call = functools.partial(
    pl.kernel,
    out_type=jax.ShapeDtypeStruct((NC * NPAD,), jnp.float32),
    mesh=_MESH,
    scratch_types=[
        pltpu.VMEM((CH,), jnp.int32),
        pltpu.VMEM((CH,), jnp.float32),
        pltpu.VMEM((ZCH,), jnp.float32),
        pltpu.VMEM_SHARED((NPAD,), jnp.float32),
        pltpu.SemaphoreType.DMA,
    ],
)(_deg_body)


def _agg_body(g_hbm, src_hbm, dst_hbm, out_hbm,
              sidx_v, didx_v, rows_v, zb_v, acc_sh, sem):
  cid = lax.axis_index("c")
  sid = lax.axis_index("s")
  wid = cid * NS + sid

  def fill_zeros(i, _):
    zb_v[i, :] = jnp.zeros((16,), jnp.float32)
    return 0

  lax.fori_loop(0, RPT, fill_zeros, 0)
  pltpu.sync_copy(zb_v, acc_sh.at[pl.ds(sid * RPT, RPT)])
  plsc.subcore_barrier()

  base = wid * EPT
  for k in range(NCHUNK):
    pltpu.sync_copy(src_hbm.at[pl.ds(base + k * CH, CH)], sidx_v)
    pltpu.sync_copy(dst_hbm.at[pl.ds(base + k * CH, CH)], didx_v)
    pltpu.async_copy(g_hbm.at[sidx_v], rows_v, sem).wait()
    pltpu.sync_copy(rows_v, acc_sh.at[didx_v], add=True)

  plsc.subcore_barrier()
  pltpu.sync_copy(acc_sh.at[pl.ds(sid * RPT, RPT)],
                  out_hbm.at[pl.ds(cid * NPAD + sid * RPT, RPT)])


_agg_call = functools.partial(
    pl.kernel,
    out_type=jax.ShapeDtypeStruct((NC * NPAD, H), jnp.float32),
    mesh=_MESH,
    scratch_types=[
        pltpu.VMEM((CH,), jnp.int32),
        pltpu.VMEM((CH,), jnp.int32),
        pltpu.VMEM((CH, H), jnp.float32),
        pltpu.VMEM((RPT, H), jnp.float32),
        pltpu.VMEM_SHARED((NPAD, H), jnp.float32),
        pltpu.SemaphoreType.DMA,
    ],
    compiler_params=pltpu.CompilerParams(use_tc_tiling_on_sc=False),
)(_agg_body)


def _tc1_body(x_ref, w1_ref, degt_ref, g1_ref, dinv_ref):
  deg = degt_ref[:, 0:1] + degt_ref[:, 1:2] + 1.0
  dinv = lax.rsqrt(deg)
  h = jnp.dot(x_ref[...], w1_ref[...],
              preferred_element_type=jnp.float32,
              precision=lax.Precision.HIGHEST)
  g1_ref[...] = h * dinv
  dinv_ref[...] = dinv


def _tc2_body(p0_ref, p1_ref, g1_ref, dinv_ref, b1_ref, g2_ref):
  dinv = dinv_ref[...]
  agg = dinv * (p0_ref[...] + p1_ref[...] + g1_ref[...]) + b1_ref[...]
  g2_ref[...] = jnp.maximum(agg, 0.0) * dinv


def _tc3_body(q0_ref, q1_ref, g2_ref, dinv_ref, w2_ref, b2_ref, out_ref):
  agg = dinv_ref[...] * (q0_ref[...] + q1_ref[...] + g2_ref[...])
  t = jnp.dot(agg, w2_ref[...],
              preferred_element_type=jnp.float32,
              precision=lax.Precision.HIGHEST) + b2_ref[...]
  m = jnp.max(t, axis=1, keepdims=True)
  e = jnp.exp(t - m)
  out_ref[...] = t - m - jnp.log(jnp.sum(e, axis=1, keepdims=True))


_tc1 = pl.pallas_call(
    _tc1_body,
    out_shape=[
        jax.ShapeDtypeStruct((N, H), jnp.float32),
        jax.ShapeDtypeStruct((N, 1), jnp.float32),
    ],
)

_tc2 = pl.pallas_call(
    _tc2_body,
    out_shape=jax.ShapeDtypeStruct((N, H), jnp.float32),
)

_tc3 = pl.pallas_call(
    _tc3_body,
    out_shape=jax.ShapeDtypeStruct((N, C), jnp.float32),
)


@jax.jit
def kernel(x, edge_index, W1, b1, W2, b2):
  src = edge_index[0]
  dst = edge_index[1]

  degp = _deg_call(dst)
  degt = degp.reshape(NC, NPAD)[:, :N].T

  g1, dinv = _tc1(x, W1, degt)

  p = _agg_call(g1, src, dst).reshape(NC, NPAD, H)[:, :N]
  g2 = _tc2(p[0], p[1], g1, dinv, b1.reshape(1, H))

  q = _agg_call(g2, src, dst).reshape(NC, NPAD, H)[:, :N]
  return _tc3(q[0], q[1], g2, dinv, W2, b2.reshape(1, C))


# speedup vs baseline: 55.0333x; 1.0981x over previous
import functools

import jax
import jax.numpy as jnp
from jax import lax
from jax.experimental import pallas as pl
from jax.experimental.pallas import tpu as pltpu
from jax.experimental.pallas import tpu_sc as plsc

N = 10000
E = 320000
D = 128
H = 16
C = 2

NC = 2
NS = 16
NW = NC * NS
EPT = E // NW
CH = 2000
NCHUNK = EPT // CH
NPAD = 10240
ZCH = NPAD // NS
RPT = NPAD // NS

_MESH = plsc.VectorSubcoreMesh(core_axis_name="c", subcore_axis_name="s")


def _deg_body(dst_hbm, out_hbm, idx0_v, idx1_v, ones_v, zer_v, acc_sh,
              sem0, sem1):
  cid = lax.axis_index("c")
  sid = lax.axis_index("s")
  wid = cid * NS + sid
  base = wid * EPT
  idx = (idx0_v, idx1_v)
  sems = (sem0, sem1)

  ld0 = pltpu.async_copy(dst_hbm.at[pl.ds(base, CH)], idx0_v, sem0)

  def fill_ones(i, _):
    ones_v[pl.ds(i * 16, 16)] = jnp.ones((16,), jnp.float32)
    return 0

  def fill_zeros(i, _):
    zer_v[pl.ds(i * 16, 16)] = jnp.zeros((16,), jnp.float32)
    return 0

  lax.fori_loop(0, CH // 16, fill_ones, 0)
  lax.fori_loop(0, ZCH // 16, fill_zeros, 0)

  pltpu.sync_copy(zer_v, acc_sh.at[pl.ds(sid * ZCH, ZCH)])
  plsc.subcore_barrier()

  loads = [None] * NCHUNK
  loads[0] = ld0
  for k in range(NCHUNK):
    loads[k].wait()
    if k + 1 < NCHUNK:
      loads[k + 1] = pltpu.async_copy(
          dst_hbm.at[pl.ds(base + (k + 1) * CH, CH)],
          idx[(k + 1) % 2], sems[(k + 1) % 2])
    pltpu.sync_copy(ones_v, acc_sh.at[idx[k % 2]], add=True)

  plsc.subcore_barrier()
  pltpu.sync_copy(acc_sh.at[pl.ds(sid * ZCH, ZCH)],
                  out_hbm.at[pl.ds(cid * NPAD + sid * ZCH, ZCH)])


_deg_call = functools.partial(
    pl.kernel,
    out_type=jax.ShapeDtypeStruct((NC * NPAD,), jnp.float32),
    mesh=_MESH,
    scratch_types=[
        pltpu.VMEM((CH,), jnp.int32),
        pltpu.VMEM((CH,), jnp.int32),
        pltpu.VMEM((CH,), jnp.float32),
        pltpu.VMEM((ZCH,), jnp.float32),
        pltpu.VMEM_SHARED((NPAD,), jnp.float32),
        pltpu.SemaphoreType.DMA,
        pltpu.SemaphoreType.DMA,
    ],
)(_deg_body)


def _agg_body(g_hbm, src_hbm, dst_hbm, out_hbm,
              sidx0, didx0, rows0, sidx1, didx1, rows1,
              zb_v, acc_sh, gsem0, gsem1, ssem0, ssem1):
  cid = lax.axis_index("c")
  sid = lax.axis_index("s")
  wid = cid * NS + sid
  base = wid * EPT
  bufs = ((sidx0, didx0, rows0, gsem0, ssem0),
          (sidx1, didx1, rows1, gsem1, ssem1))

  def load_and_gather(k):
    sidx, didx, rows, gsem, _ = bufs[k % 2]
    pltpu.sync_copy(src_hbm.at[pl.ds(base + k * CH, CH)], sidx)
    pltpu.sync_copy(dst_hbm.at[pl.ds(base + k * CH, CH)], didx)
    return pltpu.async_copy(g_hbm.at[sidx], rows, gsem)

  gd = [None] * NCHUNK
  sd = [None] * NCHUNK
  gd[0] = load_and_gather(0)

  def fill_zeros(i, _):
    zb_v[i, :] = jnp.zeros((16,), jnp.float32)
    return 0

  lax.fori_loop(0, RPT, fill_zeros, 0)
  pltpu.sync_copy(zb_v, acc_sh.at[pl.ds(sid * RPT, RPT)])
  plsc.subcore_barrier()

  for k in range(NCHUNK):
    _, didx, rows, _, ssem = bufs[k % 2]
    gd[k].wait()
    if k >= 1:
      sd[k - 1].wait()
    if k + 1 < NCHUNK:
      gd[k + 1] = load_and_gather(k + 1)
    sd[k] = pltpu.async_copy(rows, acc_sh.at[didx], ssem, add=True)
  sd[NCHUNK - 1].wait()

  plsc.subcore_barrier()
  pltpu.sync_copy(acc_sh.at[pl.ds(sid * RPT, RPT)],
                  out_hbm.at[pl.ds(cid * NPAD + sid * RPT, RPT)])


_agg_call = functools.partial(
    pl.kernel,
    out_type=jax.ShapeDtypeStruct((NC * NPAD, H), jnp.float32),
    mesh=_MESH,
    scratch_types=[
        pltpu.VMEM((CH,), jnp.int32),
        pltpu.VMEM((CH,), jnp.int32),
        pltpu.VMEM((CH, H), jnp.float32),
        pltpu.VMEM((CH,), jnp.int32),
        pltpu.VMEM((CH,), jnp.int32),
        pltpu.VMEM((CH, H), jnp.float32),
        pltpu.VMEM((RPT, H), jnp.float32),
        pltpu.VMEM_SHARED((NPAD, H), jnp.float32),
        pltpu.SemaphoreType.DMA,
        pltpu.SemaphoreType.DMA,
        pltpu.SemaphoreType.DMA,
        pltpu.SemaphoreType.DMA,
    ],
    compiler_params=pltpu.CompilerParams(use_tc_tiling_on_sc=False),
)(_agg_body)


def _tc1a_body(x_ref, w1_ref, h_ref):
  h_ref[...] = jnp.dot(x_ref[...], w1_ref[...],
                       preferred_element_type=jnp.float32,
                       precision=lax.Precision.HIGHEST)


def _tc1b_body(h_ref, degt_ref, g1_ref, dinv_ref):
  deg = degt_ref[:, 0:1] + degt_ref[:, 1:2] + 1.0
  dinv = lax.rsqrt(deg)
  g1_ref[...] = h_ref[...] * dinv
  dinv_ref[...] = dinv


def _tc2_body(p0_ref, p1_ref, g1_ref, dinv_ref, b1_ref, g2_ref):
  dinv = dinv_ref[...]
  agg = dinv * (p0_ref[...] + p1_ref[...] + g1_ref[...]) + b1_ref[...]
  g2_ref[...] = jnp.maximum(agg, 0.0) * dinv


def _tc3_body(q0_ref, q1_ref, g2_ref, dinv_ref, w2_ref, b2_ref, out_ref):
  agg = dinv_ref[...] * (q0_ref[...] + q1_ref[...] + g2_ref[...])
  t = jnp.dot(agg, w2_ref[...],
              preferred_element_type=jnp.float32,
              precision=lax.Precision.HIGHEST) + b2_ref[...]
  m = jnp.max(t, axis=1, keepdims=True)
  e = jnp.exp(t - m)
  out_ref[...] = t - m - jnp.log(jnp.sum(e, axis=1, keepdims=True))


_tc1a = pl.pallas_call(
    _tc1a_body,
    out_shape=jax.ShapeDtypeStruct((N, H), jnp.float32),
)

_tc1b = pl.pallas_call(
    _tc1b_body,
    out_shape=[
        jax.ShapeDtypeStruct((N, H), jnp.float32),
        jax.ShapeDtypeStruct((N, 1), jnp.float32),
    ],
)

_tc2 = pl.pallas_call(
    _tc2_body,
    out_shape=jax.ShapeDtypeStruct((N, H), jnp.float32),
)

_tc3 = pl.pallas_call(
    _tc3_body,
    out_shape=jax.ShapeDtypeStruct((N, C), jnp.float32),
)


@jax.jit
def kernel(x, edge_index, W1, b1, W2, b2):
  src = edge_index[0]
  dst = edge_index[1]

  h = _tc1a(x, W1)
  degp = _deg_call(dst)
  degt = degp.reshape(NC, NPAD)[:, :N].T

  g1, dinv = _tc1b(h, degt)

  p = _agg_call(g1, src, dst).reshape(NC, NPAD, H)[:, :N]
  g2 = _tc2(p[0], p[1], g1, dinv, b1.reshape(1, H))

  q = _agg_call(g2, src, dst).reshape(NC, NPAD, H)[:, :N]
  return _tc3(q[0], q[1], g2, dinv, W2, b2.reshape(1, C))


# speedup vs baseline: 66.1798x; 1.2025x over previous
import functools

import jax
import jax.numpy as jnp
from jax import lax
from jax.experimental import pallas as pl
from jax.experimental.pallas import tpu as pltpu
from jax.experimental.pallas import tpu_sc as plsc

N = 10000
E = 320000
D = 128
H = 16
C = 2

NC = 2
NS = 16
NW = NC * NS
CH = 2000
EPT = E // NW
NCHUNK = EPT // CH
EPT2 = E // NS
NCHUNK2 = EPT2 // CH
NPAD = 10240
ZCH = NPAD // NS
RPT = NPAD // NS
NPW = NPAD // NW

_MESH = plsc.VectorSubcoreMesh(core_axis_name="c", subcore_axis_name="s")


def _rsqrt16(x):
  bits = lax.bitcast_convert_type(x, jnp.int32)
  seed = jnp.full((16,), 0x5F3759DF, jnp.int32) - lax.shift_right_logical(
      bits, 1)
  y = lax.bitcast_convert_type(seed, jnp.float32)
  for _ in range(3):
    y = y * (1.5 - 0.5 * x * y * y)
  return y


def _deg_body(ei_hbm, dinv_hbm, didx_v, ones_v, zer_v, deg_v, dv_v, acc_sh,
              isem, ssem):
  cid = lax.axis_index("c")
  sid = lax.axis_index("s")
  wid = cid * NS + sid

  ld = pltpu.async_copy(ei_hbm.at[1, sid], didx_v, isem)

  def fill_ones(i, _):
    ones_v[pl.ds(i * 16, 16)] = jnp.ones((16,), jnp.float32)
    return 0

  def fill_zeros(i, _):
    zer_v[pl.ds(i * 16, 16)] = jnp.zeros((16,), jnp.float32)
    return 0

  lax.fori_loop(0, CH // 16, fill_ones, 0)
  lax.fori_loop(0, ZCH // 16, fill_zeros, 0)

  pltpu.sync_copy(zer_v, acc_sh.at[pl.ds(sid * ZCH, ZCH)])
  ld.wait()
  plsc.subcore_barrier()

  scat = [
      pltpu.async_copy(ones_v, acc_sh.at[didx_v.at[k]], ssem, add=True)
      for k in range(NCHUNK2)
  ]
  for s in scat:
    s.wait()
  plsc.subcore_barrier()

  pltpu.sync_copy(acc_sh.at[pl.ds(wid * NPW, NPW)], deg_v)

  def newton(i, _):
    d = deg_v[pl.ds(i * 16, 16)] + 1.0
    dv_v[pl.ds(i * 16, 16)] = _rsqrt16(d)
    return 0

  lax.fori_loop(0, NPW // 16, newton, 0)
  pltpu.sync_copy(dv_v, dinv_hbm.at[pl.ds(wid * NPW, NPW)])


_deg_call = functools.partial(
    pl.kernel,
    out_type=jax.ShapeDtypeStruct((NPAD,), jnp.float32),
    mesh=_MESH,
    scratch_types=[
        pltpu.VMEM((NCHUNK2, CH), jnp.int32),
        pltpu.VMEM((CH,), jnp.float32),
        pltpu.VMEM((ZCH,), jnp.float32),
        pltpu.VMEM((NPW,), jnp.float32),
        pltpu.VMEM((NPW,), jnp.float32),
        pltpu.VMEM_SHARED((NPAD,), jnp.float32),
        pltpu.SemaphoreType.DMA,
        pltpu.SemaphoreType.DMA,
    ],
    compiler_params=pltpu.CompilerParams(use_tc_tiling_on_sc=False),
)(_deg_body)


def _agg_body(g_hbm, ei_hbm, out_hbm,
              sidx_v, didx_v, rows0, rows1, zb_v, acc_sh,
              isem, gsem0, gsem1, ssem0, ssem1):
  cid = lax.axis_index("c")
  sid = lax.axis_index("s")
  wid = cid * NS + sid
  rows = (rows0, rows1)
  gsems = (gsem0, gsem1)
  ssems = (ssem0, ssem1)

  lds = pltpu.async_copy(ei_hbm.at[0, wid], sidx_v, isem)
  ldd = pltpu.async_copy(ei_hbm.at[1, wid], didx_v, isem)

  def fill_zeros(i, _):
    zb_v[i, :] = jnp.zeros((16,), jnp.float32)
    return 0

  lax.fori_loop(0, RPT, fill_zeros, 0)
  lds.wait()
  ldd.wait()

  gd = [None] * NCHUNK
  sd = [None] * NCHUNK
  gd[0] = pltpu.async_copy(g_hbm.at[sidx_v.at[0]], rows[0], gsems[0])

  pltpu.sync_copy(zb_v, acc_sh.at[pl.ds(sid * RPT, RPT)])
  plsc.subcore_barrier()

  for k in range(NCHUNK):
    gd[k].wait()
    if k >= 1:
      sd[k - 1].wait()
    if k + 1 < NCHUNK:
      gd[k + 1] = pltpu.async_copy(
          g_hbm.at[sidx_v.at[k + 1]], rows[(k + 1) % 2], gsems[(k + 1) % 2])
    sd[k] = pltpu.async_copy(
        rows[k % 2], acc_sh.at[didx_v.at[k]], ssems[k % 2], add=True)
  sd[NCHUNK - 1].wait()

  plsc.subcore_barrier()
  pltpu.sync_copy(acc_sh.at[pl.ds(sid * RPT, RPT)],
                  out_hbm.at[pl.ds(cid * NPAD + sid * RPT, RPT)])


_agg_call = functools.partial(
    pl.kernel,
    out_type=jax.ShapeDtypeStruct((NC * NPAD, H), jnp.float32),
    mesh=_MESH,
    scratch_types=[
        pltpu.VMEM((NCHUNK, CH), jnp.int32),
        pltpu.VMEM((NCHUNK, CH), jnp.int32),
        pltpu.VMEM((CH, H), jnp.float32),
        pltpu.VMEM((CH, H), jnp.float32),
        pltpu.VMEM((RPT, H), jnp.float32),
        pltpu.VMEM_SHARED((NPAD, H), jnp.float32),
        pltpu.SemaphoreType.DMA,
        pltpu.SemaphoreType.DMA,
        pltpu.SemaphoreType.DMA,
        pltpu.SemaphoreType.DMA,
        pltpu.SemaphoreType.DMA,
    ],
    compiler_params=pltpu.CompilerParams(use_tc_tiling_on_sc=False),
)(_agg_body)


def _tc1a_body(x_ref, w1_ref, h_ref):
  h_ref[0:N, :] = jnp.dot(x_ref[...], w1_ref[...],
                          preferred_element_type=jnp.float32,
                          precision=lax.Precision.HIGHEST)
  h_ref[N:NPAD, :] = jnp.zeros((NPAD - N, H), jnp.float32)


_tc1a = pl.pallas_call(
    _tc1a_body,
    out_shape=jax.ShapeDtypeStruct((NPAD, H), jnp.float32),
)

_BLK = 1280


def _tc1b_body(h_ref, dinv_ref, g1_ref):
  g1_ref[...] = h_ref[...] * dinv_ref[...]


_tc1b = pl.pallas_call(
    _tc1b_body,
    grid=(NPAD // _BLK,),
    in_specs=[
        pl.BlockSpec((_BLK, H), lambda i: (i, 0)),
        pl.BlockSpec((_BLK, 1), lambda i: (i, 0)),
    ],
    out_specs=pl.BlockSpec((_BLK, H), lambda i: (i, 0)),
    out_shape=jax.ShapeDtypeStruct((NPAD, H), jnp.float32),
)


def _tc2_body(p0_ref, p1_ref, g1_ref, dinv_ref, b1_ref, g2_ref):
  dinv = dinv_ref[...]
  agg = dinv * (p0_ref[...] + p1_ref[...] + g1_ref[...]) + b1_ref[...]
  g2_ref[...] = jnp.maximum(agg, 0.0) * dinv


_tc2 = pl.pallas_call(
    _tc2_body,
    grid=(NPAD // _BLK,),
    in_specs=[
        pl.BlockSpec((_BLK, H), lambda i: (i, 0)),
        pl.BlockSpec((_BLK, H), lambda i: (i + NPAD // _BLK, 0)),
        pl.BlockSpec((_BLK, H), lambda i: (i, 0)),
        pl.BlockSpec((_BLK, 1), lambda i: (i, 0)),
        pl.BlockSpec((1, H), lambda i: (0, 0)),
    ],
    out_specs=pl.BlockSpec((_BLK, H), lambda i: (i, 0)),
    out_shape=jax.ShapeDtypeStruct((NPAD, H), jnp.float32),
)


def _tc3_body(q_ref, g2_ref, dinv_ref, w2_ref, b2_ref, out_ref):
  agg = dinv_ref[...] * (q_ref[0:NPAD, :] + q_ref[NPAD:2 * NPAD, :]
                         + g2_ref[...])
  t = jnp.dot(agg, w2_ref[...],
              preferred_element_type=jnp.float32,
              precision=lax.Precision.HIGHEST) + b2_ref[...]
  m = jnp.max(t, axis=1, keepdims=True)
  e = jnp.exp(t - m)
  out_ref[...] = t - m - jnp.log(jnp.sum(e, axis=1, keepdims=True))


_tc3 = pl.pallas_call(
    _tc3_body,
    out_shape=jax.ShapeDtypeStruct((NPAD, C), jnp.float32),
)


@jax.jit
def kernel(x, edge_index, W1, b1, W2, b2):
  ei_agg = edge_index.reshape(2, NW, NCHUNK, CH)
  ei_deg = edge_index.reshape(2, NS, NCHUNK2, CH)

  h = _tc1a(x, W1)
  dinv = _deg_call(ei_deg)
  dinv2 = dinv.reshape(NPAD, 1)

  g1 = _tc1b(h, dinv2)
  p = _agg_call(g1, ei_agg)
  g2 = _tc2(p, p, g1, dinv2, b1.reshape(1, H))
  q = _agg_call(g2, ei_agg)
  out = _tc3(q, g2, dinv2, W2, b2.reshape(1, C))
  return out[:N]


# speedup vs baseline: 89.5219x; 1.3527x over previous
import functools

import jax
import jax.numpy as jnp
from jax import lax
from jax.experimental import pallas as pl
from jax.experimental.pallas import tpu as pltpu
from jax.experimental.pallas import tpu_sc as plsc

N = 10000
E = 320000
D = 128
H = 16
C = 2

NC = 2
NS = 16
NW = NC * NS
CH = 2000
EPT = E // NW
NCHUNK = EPT // CH
EPT2 = E // NS
NCHUNK2 = EPT2 // CH
NPAD = 10240
ZCH = NPAD // NS
RPT = NPAD // NS
NPW = NPAD // NW

_MESH = plsc.VectorSubcoreMesh(core_axis_name="c", subcore_axis_name="s")


def _rsqrt16(x):
  bits = lax.bitcast_convert_type(x, jnp.int32)
  seed = jnp.full((16,), 0x5F3759DF, jnp.int32) - lax.shift_right_logical(
      bits, 1)
  y = lax.bitcast_convert_type(seed, jnp.float32)
  for _ in range(3):
    y = y * (1.5 - 0.5 * x * y * y)
  return y


def _deg_body(ei_hbm, dinv_hbm, didx_v, ones_v, zer_v, deg_v, dv_v, acc_sh,
              isem, ssem):
  cid = lax.axis_index("c")
  sid = lax.axis_index("s")
  wid = cid * NS + sid

  ld = pltpu.async_copy(ei_hbm.at[1, sid], didx_v, isem)

  def fill_ones(i, _):
    ones_v[pl.ds(i * 16, 16)] = jnp.ones((16,), jnp.float32)
    return 0

  def fill_zeros(i, _):
    zer_v[pl.ds(i * 16, 16)] = jnp.zeros((16,), jnp.float32)
    return 0

  lax.fori_loop(0, CH // 16, fill_ones, 0)
  lax.fori_loop(0, ZCH // 16, fill_zeros, 0)

  pltpu.sync_copy(zer_v, acc_sh.at[pl.ds(sid * ZCH, ZCH)])
  ld.wait()
  plsc.subcore_barrier()

  scat = [
      pltpu.async_copy(ones_v, acc_sh.at[didx_v.at[k]], ssem, add=True)
      for k in range(NCHUNK2)
  ]
  for s in scat:
    s.wait()
  plsc.subcore_barrier()

  pltpu.sync_copy(acc_sh.at[pl.ds(wid * NPW, NPW)], deg_v)

  def newton(i, _):
    d = deg_v[pl.ds(i * 16, 16)] + 1.0
    dv_v[pl.ds(i * 16, 16)] = _rsqrt16(d)
    return 0

  lax.fori_loop(0, NPW // 16, newton, 0)
  pltpu.sync_copy(dv_v, dinv_hbm.at[pl.ds(wid * NPW, NPW)])


_deg_call = functools.partial(
    pl.kernel,
    out_type=jax.ShapeDtypeStruct((NPAD,), jnp.float32),
    mesh=_MESH,
    scratch_types=[
        pltpu.VMEM((NCHUNK2, CH), jnp.int32),
        pltpu.VMEM((CH,), jnp.float32),
        pltpu.VMEM((ZCH,), jnp.float32),
        pltpu.VMEM((NPW,), jnp.float32),
        pltpu.VMEM((NPW,), jnp.float32),
        pltpu.VMEM_SHARED((NPAD,), jnp.float32),
        pltpu.SemaphoreType.DMA,
        pltpu.SemaphoreType.DMA,
    ],
    compiler_params=pltpu.CompilerParams(use_tc_tiling_on_sc=False),
)(_deg_body)


def _agg_body(g_hbm, ei_hbm, out_hbm,
              sidx_v, didx_v, rows0, rows1, zb_v, acc_sh,
              isem, gsem0, gsem1, ssem0, ssem1):
  cid = lax.axis_index("c")
  sid = lax.axis_index("s")
  wid = cid * NS + sid
  rows = (rows0, rows1)
  gsems = (gsem0, gsem1)
  ssems = (ssem0, ssem1)

  lds = pltpu.async_copy(ei_hbm.at[0, wid], sidx_v, isem)
  ldd = pltpu.async_copy(ei_hbm.at[1, wid], didx_v, isem)

  def fill_zeros(i, _):
    zb_v[i, :] = jnp.zeros((16,), jnp.float32)
    return 0

  lax.fori_loop(0, RPT, fill_zeros, 0)
  lds.wait()
  ldd.wait()

  gd = [None] * NCHUNK
  sd = [None] * NCHUNK
  gd[0] = pltpu.async_copy(g_hbm.at[sidx_v.at[0]], rows[0], gsems[0])

  pltpu.sync_copy(zb_v, acc_sh.at[pl.ds(sid * RPT, RPT)])
  plsc.subcore_barrier()

  for k in range(NCHUNK):
    gd[k].wait()
    if k >= 1:
      sd[k - 1].wait()
    if k + 1 < NCHUNK:
      gd[k + 1] = pltpu.async_copy(
          g_hbm.at[sidx_v.at[k + 1]], rows[(k + 1) % 2], gsems[(k + 1) % 2])
    sd[k] = pltpu.async_copy(
        rows[k % 2], acc_sh.at[didx_v.at[k]], ssems[k % 2], add=True)
  sd[NCHUNK - 1].wait()

  plsc.subcore_barrier()
  pltpu.sync_copy(acc_sh.at[pl.ds(sid * RPT, RPT)],
                  out_hbm.at[pl.ds(cid * NPAD + sid * RPT, RPT)])


_agg_call = functools.partial(
    pl.kernel,
    out_type=jax.ShapeDtypeStruct((NC * NPAD, H), jnp.float32),
    mesh=_MESH,
    scratch_types=[
        pltpu.VMEM((NCHUNK, CH), jnp.int32),
        pltpu.VMEM((NCHUNK, CH), jnp.int32),
        pltpu.VMEM((CH, H), jnp.float32),
        pltpu.VMEM((CH, H), jnp.float32),
        pltpu.VMEM((RPT, H), jnp.float32),
        pltpu.VMEM_SHARED((NPAD, H), jnp.float32),
        pltpu.SemaphoreType.DMA,
        pltpu.SemaphoreType.DMA,
        pltpu.SemaphoreType.DMA,
        pltpu.SemaphoreType.DMA,
        pltpu.SemaphoreType.DMA,
    ],
    compiler_params=pltpu.CompilerParams(use_tc_tiling_on_sc=False),
)(_agg_body)


NP8 = N // 8
NPACK = NPAD // 8
_BLK = 160


def _tc1a_body(x_ref, w1b_ref, h_ref):
  h_ref[...] = jnp.dot(x_ref[...], w1b_ref[...],
                       preferred_element_type=jnp.float32,
                       precision=lax.Precision.HIGHEST)


_tc1a = pl.pallas_call(
    _tc1a_body,
    out_shape=jax.ShapeDtypeStruct((NP8, 8 * H), jnp.float32),
)


def _tc1b_body(h_ref, dinv8_ref, ones_b_ref, g1_ref, dinve_ref):
  dinve = jnp.dot(dinv8_ref[...], ones_b_ref[...],
                  preferred_element_type=jnp.float32,
                  precision=lax.Precision.HIGHEST)
  g1_ref[...] = h_ref[...] * dinve
  dinve_ref[...] = dinve


_tc1b = pl.pallas_call(
    _tc1b_body,
    grid=(NPACK // _BLK,),
    in_specs=[
        pl.BlockSpec((_BLK, 8 * H), lambda i: (i, 0)),
        pl.BlockSpec((_BLK, 8), lambda i: (i, 0)),
        pl.BlockSpec((8, 8 * H), lambda i: (0, 0)),
    ],
    out_specs=[
        pl.BlockSpec((_BLK, 8 * H), lambda i: (i, 0)),
        pl.BlockSpec((_BLK, 8 * H), lambda i: (i, 0)),
    ],
    out_shape=[
        jax.ShapeDtypeStruct((NP8, 8 * H), jnp.float32),
        jax.ShapeDtypeStruct((NP8, 8 * H), jnp.float32),
    ],
)


def _tc2_body(p0_ref, p1_ref, g1_ref, dinve_ref, b1_ref, g2_ref):
  dinve = dinve_ref[...]
  agg = dinve * (p0_ref[...] + p1_ref[...] + g1_ref[...]) + b1_ref[...]
  g2_ref[...] = jnp.maximum(agg, 0.0) * dinve


_tc2 = pl.pallas_call(
    _tc2_body,
    grid=(NPACK // _BLK,),
    in_specs=[
        pl.BlockSpec((_BLK, 8 * H), lambda i: (i, 0)),
        pl.BlockSpec((_BLK, 8 * H), lambda i: (i + NPACK // _BLK, 0)),
        pl.BlockSpec((_BLK, 8 * H), lambda i: (i, 0)),
        pl.BlockSpec((_BLK, 8 * H), lambda i: (i, 0)),
        pl.BlockSpec((1, 8 * H), lambda i: (0, 0)),
    ],
    out_specs=pl.BlockSpec((_BLK, 8 * H), lambda i: (i, 0)),
    out_shape=jax.ShapeDtypeStruct((NP8, 8 * H), jnp.float32),
)


def _tc3_body(q0_ref, q1_ref, g2_ref, dinve_ref, w2b_ref, swap_ref, b2_ref,
              out_ref):
  agg = dinve_ref[...] * (q0_ref[...] + q1_ref[...] + g2_ref[...])
  t = jnp.dot(agg, w2b_ref[...],
              preferred_element_type=jnp.float32,
              precision=lax.Precision.HIGHEST) + b2_ref[...]
  t_sw = jnp.dot(t, swap_ref[...],
                 preferred_element_type=jnp.float32,
                 precision=lax.Precision.HIGHEST)
  m = jnp.maximum(t, t_sw)
  out_ref[...] = t - m - jnp.log(jnp.exp(t - m) + jnp.exp(t_sw - m))


_tc3 = pl.pallas_call(
    _tc3_body,
    grid=(NPACK // _BLK,),
    in_specs=[
        pl.BlockSpec((_BLK, 8 * H), lambda i: (i, 0)),
        pl.BlockSpec((_BLK, 8 * H), lambda i: (i + NPACK // _BLK, 0)),
        pl.BlockSpec((_BLK, 8 * H), lambda i: (i, 0)),
        pl.BlockSpec((_BLK, 8 * H), lambda i: (i, 0)),
        pl.BlockSpec((8 * H, 8 * C), lambda i: (0, 0)),
        pl.BlockSpec((8 * C, 8 * C), lambda i: (0, 0)),
        pl.BlockSpec((1, 8 * C), lambda i: (0, 0)),
    ],
    out_specs=pl.BlockSpec((_BLK, 8 * C), lambda i: (i, 0)),
    out_shape=jax.ShapeDtypeStruct((NP8, 8 * C), jnp.float32),
)


@jax.jit
def kernel(x, edge_index, W1, b1, W2, b2):
  f32 = jnp.float32
  ei_agg = edge_index.reshape(2, NW, NCHUNK, CH)
  ei_deg = edge_index.reshape(2, NS, NCHUNK2, CH)

  eye8 = jnp.eye(8, dtype=f32)
  w1b = jnp.kron(eye8, W1)
  w2b = jnp.kron(eye8, W2)
  swap = jnp.kron(eye8, jnp.array([[0., 1.], [1., 0.]], f32))
  ones_b = jnp.kron(eye8, jnp.ones((1, H), f32))
  b1t = jnp.tile(b1, 8).reshape(1, 8 * H)
  b2t = jnp.tile(b2, 8).reshape(1, 8 * C)

  x_p = x.reshape(NP8, 8 * D)
  h_p = _tc1a(x_p, w1b)
  dinv = _deg_call(ei_deg)

  g1_p, dinve_p = _tc1b(h_p, dinv.reshape(NPACK, 8), ones_b)
  p = _agg_call(g1_p.reshape(N, H), ei_agg)
  p_p = p.reshape(2 * NPACK, 8 * H)
  g2_p = _tc2(p_p, p_p, g1_p, dinve_p, b1t)
  q = _agg_call(g2_p.reshape(N, H), ei_agg)
  q_p = q.reshape(2 * NPACK, 8 * H)
  out_p = _tc3(q_p, q_p, g2_p, dinve_p, w2b, swap, b2t)
  return out_p.reshape(N, C)


# speedup vs baseline: 93.0547x; 1.0395x over previous
import functools

import jax
import jax.numpy as jnp
from jax import lax
from jax.experimental import pallas as pl
from jax.experimental.pallas import tpu as pltpu
from jax.experimental.pallas import tpu_sc as plsc

N = 10000
E = 320000
D = 128
H = 16
C = 2

NC = 2
NS = 16
NW = NC * NS
CH = 2000
EPT = E // NW
NCHUNK = EPT // CH
EPT2 = E // NS
NCHUNK2 = EPT2 // CH
NPAD = 10240
ZCH = NPAD // NS
RPT = NPAD // NS
NPW = NPAD // NW

_MESH = plsc.VectorSubcoreMesh(core_axis_name="c", subcore_axis_name="s")


def _rsqrt16(x):
  bits = lax.bitcast_convert_type(x, jnp.int32)
  seed = jnp.full((16,), 0x5F3759DF, jnp.int32) - lax.shift_right_logical(
      bits, 1)
  y = lax.bitcast_convert_type(seed, jnp.float32)
  for _ in range(3):
    y = y * (1.5 - 0.5 * x * y * y)
  return y


def _deg_body(ei_hbm, dinv_hbm, didx_v, ones_v, zer_v, deg_v, dv_v, ev_v,
              acc_sh, isem, ssem):
  cid = lax.axis_index("c")
  sid = lax.axis_index("s")
  wid = cid * NS + sid

  ld = pltpu.async_copy(ei_hbm.at[1, sid], didx_v, isem)

  def fill_ones(i, _):
    ones_v[pl.ds(i * 16, 16)] = jnp.ones((16,), jnp.float32)
    return 0

  def fill_zeros(i, _):
    zer_v[pl.ds(i * 16, 16)] = jnp.zeros((16,), jnp.float32)
    return 0

  lax.fori_loop(0, CH // 16, fill_ones, 0)
  lax.fori_loop(0, ZCH // 16, fill_zeros, 0)

  pltpu.sync_copy(zer_v, acc_sh.at[pl.ds(sid * ZCH, ZCH)])
  ld.wait()
  plsc.subcore_barrier()

  scat = [
      pltpu.async_copy(ones_v, acc_sh.at[didx_v.at[k]], ssem, add=True)
      for k in range(NCHUNK2)
  ]
  for s in scat:
    s.wait()
  plsc.subcore_barrier()

  pltpu.sync_copy(acc_sh.at[pl.ds(wid * NPW, NPW)], deg_v)

  def newton(i, _):
    d = deg_v[pl.ds(i * 16, 16)] + 1.0
    dv_v[pl.ds(i * 16, 16)] = _rsqrt16(d)
    return 0

  lax.fori_loop(0, NPW // 16, newton, 0)

  def splat(g, _):
    d = dv_v[pl.ds(g * 16, 16)]
    for j in range(16):
      idx = jnp.full((16, 1), j, jnp.int32)
      ev_v[pl.ds(g * 256 + j * 16, 16)] = lax.gather(
          d, idx,
          lax.GatherDimensionNumbers(
              offset_dims=(), collapsed_slice_dims=(0,),
              start_index_map=(0,)),
          (1,),
          mode=lax.GatherScatterMode.PROMISE_IN_BOUNDS)
    return 0

  lax.fori_loop(0, NPW // 16, splat, 0)
  pltpu.sync_copy(ev_v, dinv_hbm.at[pl.ds(wid * NPW * 16, NPW * 16)])


_deg_call = functools.partial(
    pl.kernel,
    out_type=jax.ShapeDtypeStruct((NPAD * H,), jnp.float32),
    mesh=_MESH,
    scratch_types=[
        pltpu.VMEM((NCHUNK2, CH), jnp.int32),
        pltpu.VMEM((CH,), jnp.float32),
        pltpu.VMEM((ZCH,), jnp.float32),
        pltpu.VMEM((NPW,), jnp.float32),
        pltpu.VMEM((NPW,), jnp.float32),
        pltpu.VMEM((NPW * H,), jnp.float32),
        pltpu.VMEM_SHARED((NPAD,), jnp.float32),
        pltpu.SemaphoreType.DMA,
        pltpu.SemaphoreType.DMA,
    ],
    compiler_params=pltpu.CompilerParams(use_tc_tiling_on_sc=False),
)(_deg_body)


def _agg_body(g_hbm, ei_hbm, out_hbm,
              sidx_v, didx_v, rows0, rows1, zb_v, acc_sh,
              isem, gsem0, gsem1, ssem0, ssem1):
  cid = lax.axis_index("c")
  sid = lax.axis_index("s")
  wid = cid * NS + sid
  rows = (rows0, rows1)
  gsems = (gsem0, gsem1)
  ssems = (ssem0, ssem1)

  lds = pltpu.async_copy(ei_hbm.at[0, wid], sidx_v, isem)
  ldd = pltpu.async_copy(ei_hbm.at[1, wid], didx_v, isem)

  def fill_zeros(i, _):
    zb_v[i, :] = jnp.zeros((16,), jnp.float32)
    return 0

  lax.fori_loop(0, RPT, fill_zeros, 0)
  lds.wait()
  ldd.wait()

  gd = [None] * NCHUNK
  sd = [None] * NCHUNK
  gd[0] = pltpu.async_copy(g_hbm.at[sidx_v.at[0]], rows[0], gsems[0])

  pltpu.sync_copy(zb_v, acc_sh.at[pl.ds(sid * RPT, RPT)])
  plsc.subcore_barrier()

  for k in range(NCHUNK):
    gd[k].wait()
    if k >= 1:
      sd[k - 1].wait()
    if k + 1 < NCHUNK:
      gd[k + 1] = pltpu.async_copy(
          g_hbm.at[sidx_v.at[k + 1]], rows[(k + 1) % 2], gsems[(k + 1) % 2])
    sd[k] = pltpu.async_copy(
        rows[k % 2], acc_sh.at[didx_v.at[k]], ssems[k % 2], add=True)
  sd[NCHUNK - 1].wait()

  plsc.subcore_barrier()
  pltpu.sync_copy(acc_sh.at[pl.ds(sid * RPT, RPT)],
                  out_hbm.at[pl.ds(cid * NPAD + sid * RPT, RPT)])


_agg_call = functools.partial(
    pl.kernel,
    out_type=jax.ShapeDtypeStruct((NC * NPAD, H), jnp.float32),
    mesh=_MESH,
    scratch_types=[
        pltpu.VMEM((NCHUNK, CH), jnp.int32),
        pltpu.VMEM((NCHUNK, CH), jnp.int32),
        pltpu.VMEM((CH, H), jnp.float32),
        pltpu.VMEM((CH, H), jnp.float32),
        pltpu.VMEM((RPT, H), jnp.float32),
        pltpu.VMEM_SHARED((NPAD, H), jnp.float32),
        pltpu.SemaphoreType.DMA,
        pltpu.SemaphoreType.DMA,
        pltpu.SemaphoreType.DMA,
        pltpu.SemaphoreType.DMA,
        pltpu.SemaphoreType.DMA,
    ],
    compiler_params=pltpu.CompilerParams(use_tc_tiling_on_sc=False),
)(_agg_body)


NP8 = N // 8
NPACK = NPAD // 8
_BLK = 160
_RB = 8 * _BLK


def _tc1a_body(x_ref, w1b_ref, h_ref):
  h_ref[...] = jnp.dot(x_ref[...], w1b_ref[...],
                       preferred_element_type=jnp.float32,
                       precision=lax.Precision.HIGHEST)


_tc1a = pl.pallas_call(
    _tc1a_body,
    out_shape=jax.ShapeDtypeStruct((NP8, 8 * H), jnp.float32),
)

def _tc1b_body(h_ref, dinve_ref, g1_ref):
  g1_ref[...] = h_ref[...] * dinve_ref[...]


_tc1b = pl.pallas_call(
    _tc1b_body,
    grid=(NPACK // _BLK,),
    in_specs=[
        pl.BlockSpec((_BLK, 8 * H), lambda i: (i, 0)),
        pl.BlockSpec((_BLK, 8 * H), lambda i: (i, 0)),
    ],
    out_specs=pl.BlockSpec((_BLK, 8 * H), lambda i: (i, 0)),
    out_shape=jax.ShapeDtypeStruct((NP8, 8 * H), jnp.float32),
)


def _tc2_body(p0_ref, p1_ref, g1_ref, dinve_ref, b1_ref, g2_ref):
  dinve = dinve_ref[...]
  b1t = jnp.concatenate([b1_ref[...]] * 8, axis=1)
  agg = dinve * (p0_ref[...] + p1_ref[...] + g1_ref[...]) + b1t
  g2_ref[...] = jnp.maximum(agg, 0.0) * dinve


_tc2 = pl.pallas_call(
    _tc2_body,
    grid=(NPACK // _BLK,),
    in_specs=[
        pl.BlockSpec((_BLK, 8 * H), lambda i: (i, 0)),
        pl.BlockSpec((_BLK, 8 * H), lambda i: (i + NPACK // _BLK, 0)),
        pl.BlockSpec((_BLK, 8 * H), lambda i: (i, 0)),
        pl.BlockSpec((_BLK, 8 * H), lambda i: (i, 0)),
        pl.BlockSpec((1, H), lambda i: (0, 0)),
    ],
    out_specs=pl.BlockSpec((_BLK, 8 * H), lambda i: (i, 0)),
    out_shape=jax.ShapeDtypeStruct((NP8, 8 * H), jnp.float32),
)


def _tc3_body(q0_ref, q1_ref, g2_ref, dinve_ref, w2b_ref, b2_ref, out_ref):
  ri = lax.broadcasted_iota(jnp.int32, (8 * C, 8 * C), 0)
  ci = lax.broadcasted_iota(jnp.int32, (8 * C, 8 * C), 1)
  swap = ((ri // C == ci // C) & (ri != ci)).astype(jnp.float32)
  agg = dinve_ref[...] * (q0_ref[...] + q1_ref[...] + g2_ref[...])
  b2t = jnp.concatenate([b2_ref[...]] * 8, axis=1)
  t = jnp.dot(agg, w2b_ref[...],
              preferred_element_type=jnp.float32,
              precision=lax.Precision.HIGHEST) + b2t
  t_sw = jnp.dot(t, swap,
                 preferred_element_type=jnp.float32,
                 precision=lax.Precision.HIGHEST)
  m = jnp.maximum(t, t_sw)
  out_ref[...] = t - m - jnp.log(jnp.exp(t - m) + jnp.exp(t_sw - m))


_tc3 = pl.pallas_call(
    _tc3_body,
    grid=(NPACK // _BLK,),
    in_specs=[
        pl.BlockSpec((_BLK, 8 * H), lambda i: (i, 0)),
        pl.BlockSpec((_BLK, 8 * H), lambda i: (i + NPACK // _BLK, 0)),
        pl.BlockSpec((_BLK, 8 * H), lambda i: (i, 0)),
        pl.BlockSpec((_BLK, 8 * H), lambda i: (i, 0)),
        pl.BlockSpec((8 * H, 8 * C), lambda i: (0, 0)),
        pl.BlockSpec((1, C), lambda i: (0, 0)),
    ],
    out_specs=pl.BlockSpec((_BLK, 8 * C), lambda i: (i, 0)),
    out_shape=jax.ShapeDtypeStruct((NP8, 8 * C), jnp.float32),
)


@jax.jit
def kernel(x, edge_index, W1, b1, W2, b2):
  f32 = jnp.float32
  ei_agg = edge_index.reshape(2, NW, NCHUNK, CH)
  ei_deg = edge_index.reshape(2, NS, NCHUNK2, CH)

  eye8 = jnp.eye(8, dtype=f32)
  w1b = jnp.kron(eye8, W1)
  w2b = jnp.kron(eye8, W2)

  h_p = _tc1a(x.reshape(NP8, 8 * D), w1b)
  dinv_e = _deg_call(ei_deg)
  dinve_p = dinv_e.reshape(NPACK, 8 * H)

  g1_p = _tc1b(h_p, dinve_p)
  p = _agg_call(g1_p.reshape(N, H), ei_agg)
  p_p = p.reshape(2 * NPACK, 8 * H)
  g2_p = _tc2(p_p, p_p, g1_p, dinve_p, b1.reshape(1, H))
  q = _agg_call(g2_p.reshape(N, H), ei_agg)
  q_p = q.reshape(2 * NPACK, 8 * H)
  out_p = _tc3(q_p, q_p, g2_p, dinve_p, w2b, b2.reshape(1, C))
  return out_p.reshape(N, C)


# speedup vs baseline: 100.1190x; 1.0759x over previous
import functools

import jax
import jax.numpy as jnp
from jax import lax
from jax.experimental import pallas as pl
from jax.experimental.pallas import tpu as pltpu
from jax.experimental.pallas import tpu_sc as plsc

N = 10000
E = 320000
D = 128
H = 16
C = 2

NC = 2
NS = 16
NW = NC * NS
CH = 2000
EPT = E // NW
NCHUNK = EPT // CH
EPT2 = E // NS
NCHUNK2 = EPT2 // CH
NPAD = 10240
ZCH = NPAD // NS
RPT = NPAD // NS
NPW = NPAD // NW

_MESH = plsc.VectorSubcoreMesh(core_axis_name="c", subcore_axis_name="s")


def _rsqrt16(x):
  bits = lax.bitcast_convert_type(x, jnp.int32)
  seed = jnp.full((16,), 0x5F3759DF, jnp.int32) - lax.shift_right_logical(
      bits, 1)
  y = lax.bitcast_convert_type(seed, jnp.float32)
  for _ in range(3):
    y = y * (1.5 - 0.5 * x * y * y)
  return y


def _deg_body(ei_hbm, dinv_hbm, didx_v, ones_v, zer_v, deg_v, dv_v, ev_v,
              acc_sh, isem, ssem):
  cid = lax.axis_index("c")
  sid = lax.axis_index("s")
  wid = cid * NS + sid

  ld = pltpu.async_copy(ei_hbm.at[1, sid], didx_v, isem)

  def fill_ones(i, _):
    ones_v[pl.ds(i * 16, 16)] = jnp.ones((16,), jnp.float32)
    return 0

  def fill_zeros(i, _):
    zer_v[pl.ds(i * 16, 16)] = jnp.zeros((16,), jnp.float32)
    return 0

  lax.fori_loop(0, CH // 16, fill_ones, 0)
  lax.fori_loop(0, ZCH // 16, fill_zeros, 0)

  pltpu.sync_copy(zer_v, acc_sh.at[pl.ds(sid * ZCH, ZCH)])
  ld.wait()
  plsc.subcore_barrier()

  scat = [
      pltpu.async_copy(ones_v, acc_sh.at[didx_v.at[k]], ssem, add=True)
      for k in range(NCHUNK2)
  ]
  for s in scat:
    s.wait()
  plsc.subcore_barrier()

  pltpu.sync_copy(acc_sh.at[pl.ds(wid * NPW, NPW)], deg_v)

  def newton(i, _):
    d = deg_v[pl.ds(i * 16, 16)] + 1.0
    dv_v[pl.ds(i * 16, 16)] = _rsqrt16(d)
    return 0

  lax.fori_loop(0, NPW // 16, newton, 0)

  def splat(g, _):
    d = dv_v[pl.ds(g * 16, 16)]
    for j in range(16):
      idx = jnp.full((16, 1), j, jnp.int32)
      ev_v[pl.ds(g * 256 + j * 16, 16)] = lax.gather(
          d, idx,
          lax.GatherDimensionNumbers(
              offset_dims=(), collapsed_slice_dims=(0,),
              start_index_map=(0,)),
          (1,),
          mode=lax.GatherScatterMode.PROMISE_IN_BOUNDS)
    return 0

  lax.fori_loop(0, NPW // 16, splat, 0)
  pltpu.sync_copy(ev_v, dinv_hbm.at[pl.ds(wid * NPW * 16, NPW * 16)])


_deg_call = functools.partial(
    pl.kernel,
    out_type=jax.ShapeDtypeStruct((NPAD * H,), jnp.float32),
    mesh=_MESH,
    scratch_types=[
        pltpu.VMEM((NCHUNK2, CH), jnp.int32),
        pltpu.VMEM((CH,), jnp.float32),
        pltpu.VMEM((ZCH,), jnp.float32),
        pltpu.VMEM((NPW,), jnp.float32),
        pltpu.VMEM((NPW,), jnp.float32),
        pltpu.VMEM((NPW * H,), jnp.float32),
        pltpu.VMEM_SHARED((NPAD,), jnp.float32),
        pltpu.SemaphoreType.DMA,
        pltpu.SemaphoreType.DMA,
    ],
    compiler_params=pltpu.CompilerParams(use_tc_tiling_on_sc=False),
)(_deg_body)


def _agg_body(g_hbm, ei_hbm, out_hbm,
              sidx_v, didx_v, rows0, rows1, zb_v, acc_sh, tbl_sh,
              isem, tsem, gsem0, gsem1, ssem0, ssem1):
  cid = lax.axis_index("c")
  sid = lax.axis_index("s")
  wid = cid * NS + sid
  rows = (rows0, rows1)
  gsems = (gsem0, gsem1)
  ssems = (ssem0, ssem1)

  lds = pltpu.async_copy(ei_hbm.at[0, wid], sidx_v, isem)
  ldd = pltpu.async_copy(ei_hbm.at[1, wid], didx_v, isem)
  NT = N // NS
  ldt = pltpu.async_copy(g_hbm.at[pl.ds(sid * NT, NT)],
                         tbl_sh.at[pl.ds(sid * NT, NT)], tsem)

  def fill_zeros(i, _):
    zb_v[i, :] = jnp.zeros((16,), jnp.float32)
    return 0

  lax.fori_loop(0, RPT, fill_zeros, 0)
  pltpu.sync_copy(zb_v, acc_sh.at[pl.ds(sid * RPT, RPT)])
  lds.wait()
  ldd.wait()
  ldt.wait()
  plsc.subcore_barrier()

  gd = [None] * NCHUNK
  sd = [None] * NCHUNK
  gd[0] = pltpu.async_copy(tbl_sh.at[sidx_v.at[0]], rows[0], gsems[0])

  for k in range(NCHUNK):
    gd[k].wait()
    if k >= 1:
      sd[k - 1].wait()
    if k + 1 < NCHUNK:
      gd[k + 1] = pltpu.async_copy(
          tbl_sh.at[sidx_v.at[k + 1]], rows[(k + 1) % 2], gsems[(k + 1) % 2])
    sd[k] = pltpu.async_copy(
        rows[k % 2], acc_sh.at[didx_v.at[k]], ssems[k % 2], add=True)
  sd[NCHUNK - 1].wait()

  plsc.subcore_barrier()
  pltpu.sync_copy(acc_sh.at[pl.ds(sid * RPT, RPT)],
                  out_hbm.at[pl.ds(cid * NPAD + sid * RPT, RPT)])


_agg_call = functools.partial(
    pl.kernel,
    out_type=jax.ShapeDtypeStruct((NC * NPAD, H), jnp.float32),
    mesh=_MESH,
    scratch_types=[
        pltpu.VMEM((NCHUNK, CH), jnp.int32),
        pltpu.VMEM((NCHUNK, CH), jnp.int32),
        pltpu.VMEM((CH, H), jnp.float32),
        pltpu.VMEM((CH, H), jnp.float32),
        pltpu.VMEM((RPT, H), jnp.float32),
        pltpu.VMEM_SHARED((NPAD, H), jnp.float32),
        pltpu.VMEM_SHARED((N, H), jnp.float32),
        pltpu.SemaphoreType.DMA,
        pltpu.SemaphoreType.DMA,
        pltpu.SemaphoreType.DMA,
        pltpu.SemaphoreType.DMA,
        pltpu.SemaphoreType.DMA,
        pltpu.SemaphoreType.DMA,
    ],
    compiler_params=pltpu.CompilerParams(use_tc_tiling_on_sc=False),
)(_agg_body)


NP8 = N // 8
NPACK = NPAD // 8
_BLK = 160
_RB = 8 * _BLK


def _tc1a_body(x_ref, w1b_ref, h_ref):
  h_ref[...] = jnp.dot(x_ref[...], w1b_ref[...],
                       preferred_element_type=jnp.float32,
                       precision=lax.Precision.HIGHEST)


_tc1a = pl.pallas_call(
    _tc1a_body,
    out_shape=jax.ShapeDtypeStruct((NP8, 8 * H), jnp.float32),
)

def _tc1b_body(h_ref, dinve_ref, g1_ref):
  g1_ref[...] = h_ref[...] * dinve_ref[...]


_tc1b = pl.pallas_call(
    _tc1b_body,
    grid=(NPACK // _BLK,),
    in_specs=[
        pl.BlockSpec((_BLK, 8 * H), lambda i: (i, 0)),
        pl.BlockSpec((_BLK, 8 * H), lambda i: (i, 0)),
    ],
    out_specs=pl.BlockSpec((_BLK, 8 * H), lambda i: (i, 0)),
    out_shape=jax.ShapeDtypeStruct((NP8, 8 * H), jnp.float32),
)


def _tc2_body(p0_ref, p1_ref, g1_ref, dinve_ref, b1_ref, g2_ref):
  dinve = dinve_ref[...]
  b1t = jnp.concatenate([b1_ref[...]] * 8, axis=1)
  agg = dinve * (p0_ref[...] + p1_ref[...] + g1_ref[...]) + b1t
  g2_ref[...] = jnp.maximum(agg, 0.0) * dinve


_tc2 = pl.pallas_call(
    _tc2_body,
    grid=(NPACK // _BLK,),
    in_specs=[
        pl.BlockSpec((_BLK, 8 * H), lambda i: (i, 0)),
        pl.BlockSpec((_BLK, 8 * H), lambda i: (i + NPACK // _BLK, 0)),
        pl.BlockSpec((_BLK, 8 * H), lambda i: (i, 0)),
        pl.BlockSpec((_BLK, 8 * H), lambda i: (i, 0)),
        pl.BlockSpec((1, H), lambda i: (0, 0)),
    ],
    out_specs=pl.BlockSpec((_BLK, 8 * H), lambda i: (i, 0)),
    out_shape=jax.ShapeDtypeStruct((NP8, 8 * H), jnp.float32),
)


def _tc3_body(q0_ref, q1_ref, g2_ref, dinve_ref, w2b_ref, b2_ref, out_ref):
  ri = lax.broadcasted_iota(jnp.int32, (8 * C, 8 * C), 0)
  ci = lax.broadcasted_iota(jnp.int32, (8 * C, 8 * C), 1)
  swap = ((ri // C == ci // C) & (ri != ci)).astype(jnp.float32)
  agg = dinve_ref[...] * (q0_ref[...] + q1_ref[...] + g2_ref[...])
  b2t = jnp.concatenate([b2_ref[...]] * 8, axis=1)
  t = jnp.dot(agg, w2b_ref[...],
              preferred_element_type=jnp.float32,
              precision=lax.Precision.HIGHEST) + b2t
  t_sw = jnp.dot(t, swap,
                 preferred_element_type=jnp.float32,
                 precision=lax.Precision.HIGHEST)
  m = jnp.maximum(t, t_sw)
  out_ref[...] = t - m - jnp.log(jnp.exp(t - m) + jnp.exp(t_sw - m))


_tc3 = pl.pallas_call(
    _tc3_body,
    grid=(NPACK // _BLK,),
    in_specs=[
        pl.BlockSpec((_BLK, 8 * H), lambda i: (i, 0)),
        pl.BlockSpec((_BLK, 8 * H), lambda i: (i + NPACK // _BLK, 0)),
        pl.BlockSpec((_BLK, 8 * H), lambda i: (i, 0)),
        pl.BlockSpec((_BLK, 8 * H), lambda i: (i, 0)),
        pl.BlockSpec((8 * H, 8 * C), lambda i: (0, 0)),
        pl.BlockSpec((1, C), lambda i: (0, 0)),
    ],
    out_specs=pl.BlockSpec((_BLK, 8 * C), lambda i: (i, 0)),
    out_shape=jax.ShapeDtypeStruct((NP8, 8 * C), jnp.float32),
)


@jax.jit
def kernel(x, edge_index, W1, b1, W2, b2):
  f32 = jnp.float32
  ei_agg = edge_index.reshape(2, NW, NCHUNK, CH)
  ei_deg = edge_index.reshape(2, NS, NCHUNK2, CH)

  eye8 = jnp.eye(8, dtype=f32)
  w1b = jnp.kron(eye8, W1)
  w2b = jnp.kron(eye8, W2)

  h_p = _tc1a(x.reshape(NP8, 8 * D), w1b)
  dinv_e = _deg_call(ei_deg)
  dinve_p = dinv_e.reshape(NPACK, 8 * H)

  g1_p = _tc1b(h_p, dinve_p)
  p = _agg_call(g1_p.reshape(N, H), ei_agg)
  p_p = p.reshape(2 * NPACK, 8 * H)
  g2_p = _tc2(p_p, p_p, g1_p, dinve_p, b1.reshape(1, H))
  q = _agg_call(g2_p.reshape(N, H), ei_agg)
  q_p = q.reshape(2 * NPACK, 8 * H)
  out_p = _tc3(q_p, q_p, g2_p, dinve_p, w2b, b2.reshape(1, C))
  return out_p.reshape(N, C)
